# baseline (speedup 1.0000x reference)
import functools

import jax
import jax.numpy as jnp
from jax import lax
from jax.experimental import pallas as pl
from jax.experimental.pallas import tpu as pltpu
from jax.experimental.pallas import tpu_sc as plsc

NC = 2
NS = 16
NW = NC * NS
CH = 128


@functools.lru_cache(maxsize=None)
def _make_gather(n_nodes, n_edges, d):
    n_chunks = n_edges // CH
    per_w = (n_chunks + NW - 1) // NW
    mesh = plsc.VectorSubcoreMesh(core_axis_name="c", subcore_axis_name="s")

    @functools.partial(
        pl.kernel,
        mesh=mesh,
        out_type=[jax.ShapeDtypeStruct((n_edges, d), jnp.float32),
                  jax.ShapeDtypeStruct((n_edges, d), jnp.float32)],
        scratch_types=[
            pltpu.VMEM((CH,), jnp.int32),
            pltpu.VMEM((CH,), jnp.int32),
            pltpu.VMEM((CH, d), jnp.float32),
            pltpu.VMEM((CH, d), jnp.float32),
            pltpu.SemaphoreType.DMA,
            pltpu.SemaphoreType.DMA,
        ],
        compiler_params=pltpu.CompilerParams(use_tc_tiling_on_sc=False),
    )
    def gather_k(x_hbm, row_hbm, col_hbm, o_row, o_col,
                 idx_r, idx_c, buf_r, buf_c, sem_r, sem_c):
        c = lax.axis_index("c")
        s = lax.axis_index("s")
        wid = s * NC + c

        def body(t, carry):
            ch = t * NW + wid

            @pl.when(ch < n_chunks)
            def _():
                base = ch * CH
                pltpu.sync_copy(row_hbm.at[pl.ds(base, CH)], idx_r)
                pltpu.sync_copy(col_hbm.at[pl.ds(base, CH)], idx_c)
                cp_r = pltpu.async_copy(x_hbm.at[idx_r], buf_r, sem_r)
                cp_c = pltpu.async_copy(x_hbm.at[idx_c], buf_c, sem_c)
                cp_r.wait()
                cp_c.wait()
                pltpu.sync_copy(buf_r, o_row.at[pl.ds(base, CH)])
                pltpu.sync_copy(buf_c, o_col.at[pl.ds(base, CH)])

            return carry

        lax.fori_loop(0, per_w, body, 0)

    return gather_k


@functools.lru_cache(maxsize=None)
def _make_scatter(n_nodes, n_edges, d):
    n_chunks = n_edges // CH
    per_w = (n_chunks + NW - 1) // NW
    lp = min(d, 128)
    n_pass = d // lp
    nr = n_nodes // NS
    zr = 125
    mesh = plsc.VectorSubcoreMesh(core_axis_name="c", subcore_axis_name="s")

    @functools.partial(
        pl.kernel,
        mesh=mesh,
        out_type=[jax.ShapeDtypeStruct((n_nodes, d), jnp.float32),
                  jax.ShapeDtypeStruct((n_nodes, d), jnp.float32)],
        scratch_types=[
            pltpu.VMEM((CH,), jnp.int32),
            pltpu.VMEM((CH, lp), jnp.float32),
            pltpu.VMEM((zr, lp), jnp.float32),
            pltpu.VMEM_SHARED((n_nodes, lp), jnp.float32),
        ],
        compiler_params=pltpu.CompilerParams(use_tc_tiling_on_sc=False),
    )
    def scatter_k(h_hbm, col_hbm, out0, out1, idx_v, hbuf, zbuf, acc):
        c = lax.axis_index("c")
        s = lax.axis_index("s")
        wid = s * NC + c

        def zrow(r, carry):
            for k in range(lp // 16):
                zbuf[r, pl.ds(k * 16, 16)] = jnp.zeros((16,), jnp.float32)
            return carry

        lax.fori_loop(0, zr, zrow, 0)

        for p in range(n_pass):
            for z in range(nr // zr):
                pltpu.sync_copy(zbuf, acc.at[pl.ds(s * nr + z * zr, zr)])
            plsc.subcore_barrier()

            def body(t, carry):
                ch = t * NW + wid

                @pl.when(ch < n_chunks)
                def _():
                    base = ch * CH
                    pltpu.sync_copy(col_hbm.at[pl.ds(base, CH)], idx_v)
                    pltpu.sync_copy(
                        h_hbm.at[pl.ds(base, CH), pl.ds(p * lp, lp)], hbuf)
                    pltpu.sync_copy(hbuf, acc.at[idx_v], add=True)

                return carry

            lax.fori_loop(0, per_w, body, 0)
            plsc.subcore_barrier()

            @pl.when(c == 0)
            def _():
                pltpu.sync_copy(acc.at[pl.ds(s * nr, nr)],
                                out0.at[pl.ds(s * nr, nr), pl.ds(p * lp, lp)])

            @pl.when(c == 1)
            def _():
                pltpu.sync_copy(acc.at[pl.ds(s * nr, nr)],
                                out1.at[pl.ds(s * nr, nr), pl.ds(p * lp, lp)])

            if p + 1 < n_pass:
                plsc.subcore_barrier()

    return scatter_k


def _pick_br(rows):
    if rows <= 16384 and rows % 8 == 0:
        return rows
    for br in (4000, 2000, 1000):
        if rows % br == 0:
            return br
    raise ValueError(rows)


def _fused_mm(parts, wts, bias, act, with_stats=True):
    rows = parts[0].shape[0]
    dout = wts[0].shape[1]
    np_ = len(parts)
    _BR = _pick_br(rows)
    grid = rows // _BR

    def kern(*refs):
        i = pl.program_id(0)
        xs = refs[:np_]
        ws = refs[np_:2 * np_]
        b = refs[2 * np_]
        y = refs[2 * np_ + 1]
        acc = jax.lax.dot_general(xs[0][...], ws[0][...],
                                  (((1,), (0,)), ((), ())),
                                  preferred_element_type=jnp.float32)
        for k in range(1, np_):
            acc = acc + jax.lax.dot_general(xs[k][...], ws[k][...],
                                            (((1,), (0,)), ((), ())),
                                            preferred_element_type=jnp.float32)
        acc = acc + b[...]
        if act == "leaky":
            acc = jnp.where(acc >= 0, acc, 0.1 * acc)
        elif act == "sigmoid":
            acc = jax.nn.sigmoid(acc)
        y[...] = acc
        if with_stats:
            st = refs[2 * np_ + 2]

            @pl.when(i == 0)
            def _():
                st[...] = jnp.zeros_like(st)

            st[...] += jnp.concatenate(
                [jnp.sum(acc, 0, keepdims=True),
                 jnp.sum(acc * acc, 0, keepdims=True)], 0)

    in_specs = (
        [pl.BlockSpec((_BR, p.shape[1]), lambda i: (i, 0)) for p in parts]
        + [pl.BlockSpec(w.shape, lambda i: (0, 0)) for w in wts]
        + [pl.BlockSpec((1, dout), lambda i: (0, 0))]
    )
    out_specs = [pl.BlockSpec((_BR, dout), lambda i: (i, 0))]
    out_shape = [jax.ShapeDtypeStruct((rows, dout), jnp.float32)]
    if with_stats:
        out_specs.append(pl.BlockSpec((2, dout), lambda i: (0, 0)))
        out_shape.append(jax.ShapeDtypeStruct((2, dout), jnp.float32))

    res = pl.pallas_call(
        kern, grid=(grid,), in_specs=in_specs,
        out_specs=out_specs, out_shape=out_shape,
    )(*parts, *wts, bias)
    if with_stats:
        return res[0], res[1]
    return res[0]


def _colstats_pair(xa, xb):
    rows, d = xa.shape
    _BR = _pick_br(rows)
    grid = rows // _BR

    def kern(ra, rb, sa, sb):
        i = pl.program_id(0)

        @pl.when(i == 0)
        def _():
            sa[...] = jnp.zeros_like(sa)
            sb[...] = jnp.zeros_like(sb)

        va = ra[...]
        vb = rb[...]
        sa[...] += jnp.concatenate(
            [jnp.sum(va, 0, keepdims=True),
             jnp.sum(va * va, 0, keepdims=True)], 0)
        sb[...] += jnp.concatenate(
            [jnp.sum(vb, 0, keepdims=True),
             jnp.sum(vb * vb, 0, keepdims=True)], 0)

    return pl.pallas_call(
        kern, grid=(grid,),
        in_specs=[pl.BlockSpec((_BR, d), lambda i: (i, 0)),
                  pl.BlockSpec((_BR, d), lambda i: (i, 0))],
        out_specs=[pl.BlockSpec((2, d), lambda i: (0, 0)),
                   pl.BlockSpec((2, d), lambda i: (0, 0))],
        out_shape=[jax.ShapeDtypeStruct((2, d), jnp.float32),
                   jax.ShapeDtypeStruct((2, d), jnp.float32)],
    )(xa, xb)


def _colstats(*parts):
    rows, d = parts[0].shape
    np_ = len(parts)
    _BR = _pick_br(rows)
    grid = rows // _BR

    def kern(*refs):
        i = pl.program_id(0)
        x = refs[0][...]
        for k in range(1, np_):
            x = x + refs[k][...]
        st = refs[np_]

        @pl.when(i == 0)
        def _():
            st[...] = jnp.zeros_like(st)

        st[...] += jnp.concatenate(
            [jnp.sum(x, 0, keepdims=True),
             jnp.sum(x * x, 0, keepdims=True)], 0)

    return pl.pallas_call(
        kern, grid=(grid,),
        in_specs=[pl.BlockSpec((_BR, d), lambda i: (i, 0)) for _ in parts],
        out_specs=pl.BlockSpec((2, d), lambda i: (0, 0)),
        out_shape=jax.ShapeDtypeStruct((2, d), jnp.float32),
    )(*parts)


def _fold(bn, lin, stats_list, part_dims, rows):
    st = jnp.concatenate(stats_list, axis=1)
    mean = st[0] / rows
    var = jnp.maximum(st[1] / rows - mean * mean, 0.0)
    a = bn["g"] * lax.rsqrt(var + 1e-5)
    shift = bn["b"] - mean * a
    wt = lin["W"].T * a[:, None]
    beff = (lin["b"] + shift @ lin["W"].T).reshape(1, -1)
    w_parts = []
    off = 0
    for pd in part_dims:
        w_parts.append(wt[off:off + pd])
        off += pd
    return w_parts, beff


def _mlp3(parts, stats_list, mlp, rows, last_act="none"):
    part_dims = [p.shape[1] for p in parts]
    w0, b0 = _fold(mlp["bn0"], mlp["lin0"], stats_list, part_dims, rows)
    a1, st1 = _fused_mm(parts, w0, b0, "leaky")
    w1, b1 = _fold(mlp["bn1"], mlp["lin1"], [st1], [a1.shape[1]], rows)
    a2, st2 = _fused_mm([a1], w1, b1, "leaky")
    w2, b2 = _fold(mlp["bn2"], mlp["lin2"], [st2], [a2.shape[1]], rows)
    return _fused_mm([a2], w2, b2, last_act if last_act != "none" else "none")


def kernel(x, edge_index, edge_attr, u, batch, params):
    n_nodes, d_node = x.shape
    n_edges, d_edge = edge_attr.shape
    row = edge_index[0]
    col = edge_index[1]

    gather = _make_gather(n_nodes, n_edges, d_node)
    x_st = _colstats(x)
    ea_st = _colstats(edge_attr)

    for name in ("ml1", "ml2", "ml3"):
        p = params[name]
        hid = p["node1"]["lin2"]["b"].shape[0]

        xs, xd = gather(x, row, col)
        xs_st, xd_st = _colstats_pair(xs, xd)

        edge_attr, ea_st = _mlp3([xs, xd, edge_attr],
                                 [xs_st, xd_st, ea_st],
                                 p["edge"], n_edges)

        h, _ = _mlp3([xs, edge_attr], [xs_st, ea_st],
                     p["node1"], n_edges)

        scatter = _make_scatter(n_nodes, n_edges, hid)
        agg0, agg1 = scatter(h, col)
        agg_st = _colstats(agg0, agg1)

        part_dims = [d_node, hid]
        w0, b0 = _fold(p["node2"]["bn0"], p["node2"]["lin0"],
                       [x_st, agg_st], part_dims, n_nodes)
        a1, st1 = _fused_mm([x, agg0, agg1], [w0[0], w0[1], w0[1]],
                            b0, "leaky")
        w1, b1 = _fold(p["node2"]["bn1"], p["node2"]["lin1"],
                       [st1], [a1.shape[1]], n_nodes)
        a2, st2 = _fused_mm([a1], w1, b1, "leaky")
        w2, b2 = _fold(p["node2"]["bn2"], p["node2"]["lin2"],
                       [st2], [a2.shape[1]], n_nodes)
        x, x_st = _fused_mm([a2], w2, b2, "none")

    y_pred = _fused_mm([x], [params["x_linear"]["W"].T],
                       params["x_linear"]["b"].reshape(1, -1),
                       "sigmoid", with_stats=False)
    edge_label_pred = _fused_mm([edge_attr], [params["edge_linear"]["W"].T],
                                params["edge_linear"]["b"].reshape(1, -1),
                                "sigmoid", with_stats=False)
    return (y_pred, edge_label_pred)


# speedup vs baseline: 1.0678x; 1.0678x over previous
import functools

import jax
import jax.numpy as jnp
from jax import lax
from jax.experimental import pallas as pl
from jax.experimental.pallas import tpu as pltpu
from jax.experimental.pallas import tpu_sc as plsc

NC = 2
NS = 16
NW = NC * NS
CH = 128


@functools.lru_cache(maxsize=None)
def _make_gather(n_nodes, n_edges, d):
    n_chunks = n_edges // CH
    nu = 4
    per_w = (n_chunks + NW - 1) // NW
    n_win = (per_w + nu - 1) // nu
    mesh = plsc.VectorSubcoreMesh(core_axis_name="c", subcore_axis_name="s")

    scratch = []
    for _ in range(nu):
        scratch += [pltpu.VMEM((CH,), jnp.int32), pltpu.VMEM((CH,), jnp.int32),
                    pltpu.VMEM((CH, d), jnp.float32),
                    pltpu.VMEM((CH, d), jnp.float32),
                    pltpu.SemaphoreType.DMA, pltpu.SemaphoreType.DMA,
                    pltpu.SemaphoreType.DMA]

    @functools.partial(
        pl.kernel,
        mesh=mesh,
        out_type=[jax.ShapeDtypeStruct((n_edges, d), jnp.float32),
                  jax.ShapeDtypeStruct((n_edges, d), jnp.float32)],
        scratch_types=scratch,
        compiler_params=pltpu.CompilerParams(use_tc_tiling_on_sc=False),
    )
    def gather_k(x_hbm, row_hbm, col_hbm, o_row, o_col, *bufs):
        c = lax.axis_index("c")
        s = lax.axis_index("s")
        wid = s * NC + c
        sets = [bufs[7 * k:7 * k + 7] for k in range(nu)]

        def window(j, carry):
            def chunk(k):
                return (j * nu + k) * NW + wid

            for k in range(nu):
                idx_r, idx_c, buf_r, buf_c, sem_i, sem_g, sem_w = sets[k]
                ch = chunk(k)

                @pl.when(ch < n_chunks)
                def _(ch=ch, idx_r=idx_r, idx_c=idx_c, sem_i=sem_i):
                    base = ch * CH
                    pltpu.async_copy(row_hbm.at[pl.ds(base, CH)], idx_r, sem_i)
                    pltpu.async_copy(col_hbm.at[pl.ds(base, CH)], idx_c, sem_i)

            for k in range(nu):
                idx_r, idx_c, buf_r, buf_c, sem_i, sem_g, sem_w = sets[k]
                ch = chunk(k)

                @pl.when(ch < n_chunks)
                def _(ch=ch, idx_r=idx_r, idx_c=idx_c, buf_r=buf_r,
                      buf_c=buf_c, sem_i=sem_i, sem_g=sem_g):
                    base = ch * CH
                    pltpu.make_async_copy(
                        row_hbm.at[pl.ds(base, CH)], idx_r, sem_i).wait()
                    pltpu.make_async_copy(
                        col_hbm.at[pl.ds(base, CH)], idx_c, sem_i).wait()
                    pltpu.async_copy(x_hbm.at[idx_r], buf_r, sem_g)
                    pltpu.async_copy(x_hbm.at[idx_c], buf_c, sem_g)

            for k in range(nu):
                idx_r, idx_c, buf_r, buf_c, sem_i, sem_g, sem_w = sets[k]
                ch = chunk(k)

                @pl.when(ch < n_chunks)
                def _(ch=ch, idx_r=idx_r, idx_c=idx_c, buf_r=buf_r,
                      buf_c=buf_c, sem_g=sem_g, sem_w=sem_w):
                    base = ch * CH
                    pltpu.make_async_copy(x_hbm.at[idx_r], buf_r, sem_g).wait()
                    pltpu.make_async_copy(x_hbm.at[idx_c], buf_c, sem_g).wait()
                    pltpu.async_copy(buf_r, o_row.at[pl.ds(base, CH)], sem_w)
                    pltpu.async_copy(buf_c, o_col.at[pl.ds(base, CH)], sem_w)

            for k in range(nu):
                idx_r, idx_c, buf_r, buf_c, sem_i, sem_g, sem_w = sets[k]
                ch = chunk(k)

                @pl.when(ch < n_chunks)
                def _(ch=ch, buf_r=buf_r, buf_c=buf_c, sem_w=sem_w):
                    base = ch * CH
                    pltpu.make_async_copy(
                        buf_r, o_row.at[pl.ds(base, CH)], sem_w).wait()
                    pltpu.make_async_copy(
                        buf_c, o_col.at[pl.ds(base, CH)], sem_w).wait()

            return carry

        lax.fori_loop(0, n_win, window, 0)

    return gather_k


@functools.lru_cache(maxsize=None)
def _make_scatter(n_nodes, n_edges, d):
    n_chunks = n_edges // CH
    nu = 2
    per_w = (n_chunks + NW - 1) // NW
    n_win = (per_w + nu - 1) // nu
    lp = min(d, 128)
    n_pass = d // lp
    nr = n_nodes // NS
    zr = 25
    mesh = plsc.VectorSubcoreMesh(core_axis_name="c", subcore_axis_name="s")

    @functools.partial(
        pl.kernel,
        mesh=mesh,
        out_type=[jax.ShapeDtypeStruct((n_nodes, d), jnp.float32),
                  jax.ShapeDtypeStruct((n_nodes, d), jnp.float32)],
        scratch_types=(
            [pltpu.VMEM((zr, lp), jnp.float32),
             pltpu.VMEM_SHARED((n_nodes, lp), jnp.float32)]
            + [pltpu.VMEM((CH,), jnp.int32), pltpu.VMEM((CH, lp), jnp.float32),
               pltpu.SemaphoreType.DMA, pltpu.SemaphoreType.DMA,
               pltpu.SemaphoreType.DMA] * 2
        ),
        compiler_params=pltpu.CompilerParams(use_tc_tiling_on_sc=False),
    )
    def scatter_k(h_hbm, col_hbm, out0, out1, zbuf, acc, *bufs):
        c = lax.axis_index("c")
        s = lax.axis_index("s")
        wid = s * NC + c
        sets = [bufs[5 * k:5 * k + 5] for k in range(nu)]

        def zrow(r, carry):
            for k in range(lp // 16):
                zbuf[r, pl.ds(k * 16, 16)] = jnp.zeros((16,), jnp.float32)
            return carry

        lax.fori_loop(0, zr, zrow, 0)

        for p in range(n_pass):
            for z in range(nr // zr):
                pltpu.sync_copy(zbuf, acc.at[pl.ds(s * nr + z * zr, zr)])
            plsc.subcore_barrier()

            def window(j, carry):
                def chunk(k):
                    return (j * nu + k) * NW + wid

                for k in range(nu):
                    idx_v, hbuf, sem_i, sem_h, sem_s = sets[k]
                    ch = chunk(k)

                    @pl.when(ch < n_chunks)
                    def _(ch=ch, idx_v=idx_v, hbuf=hbuf,
                          sem_i=sem_i, sem_h=sem_h):
                        base = ch * CH
                        pltpu.async_copy(
                            col_hbm.at[pl.ds(base, CH)], idx_v, sem_i)
                        pltpu.async_copy(
                            h_hbm.at[pl.ds(base, CH), pl.ds(p * lp, lp)],
                            hbuf, sem_h)

                for k in range(nu):
                    idx_v, hbuf, sem_i, sem_h, sem_s = sets[k]
                    ch = chunk(k)

                    @pl.when(ch < n_chunks)
                    def _(ch=ch, idx_v=idx_v, hbuf=hbuf, sem_i=sem_i,
                          sem_h=sem_h, sem_s=sem_s):
                        base = ch * CH
                        pltpu.make_async_copy(
                            col_hbm.at[pl.ds(base, CH)], idx_v, sem_i).wait()
                        pltpu.make_async_copy(
                            h_hbm.at[pl.ds(base, CH), pl.ds(p * lp, lp)],
                            hbuf, sem_h).wait()
                        pltpu.async_copy(hbuf, acc.at[idx_v], sem_s, add=True)

                for k in range(nu):
                    idx_v, hbuf, sem_i, sem_h, sem_s = sets[k]
                    ch = chunk(k)

                    @pl.when(ch < n_chunks)
                    def _(ch=ch, idx_v=idx_v, hbuf=hbuf, sem_s=sem_s):
                        pltpu.make_async_copy(
                            hbuf, acc.at[idx_v], sem_s).wait()

                return carry

            lax.fori_loop(0, n_win, window, 0)
            plsc.subcore_barrier()

            @pl.when(c == 0)
            def _():
                pltpu.sync_copy(acc.at[pl.ds(s * nr, nr)],
                                out0.at[pl.ds(s * nr, nr), pl.ds(p * lp, lp)])

            @pl.when(c == 1)
            def _():
                pltpu.sync_copy(acc.at[pl.ds(s * nr, nr)],
                                out1.at[pl.ds(s * nr, nr), pl.ds(p * lp, lp)])

            if p + 1 < n_pass:
                plsc.subcore_barrier()

    return scatter_k


def _pick_br(rows):
    if rows <= 16384 and rows % 8 == 0:
        return rows
    for br in (4000, 2000, 1000):
        if rows % br == 0:
            return br
    raise ValueError(rows)


def _fused_mm(parts, wts, bias, act, with_stats=True):
    rows = parts[0].shape[0]
    dout = wts[0].shape[1]
    np_ = len(parts)
    _BR = _pick_br(rows)
    grid = rows // _BR

    def kern(*refs):
        i = pl.program_id(0)
        xs = refs[:np_]
        ws = refs[np_:2 * np_]
        b = refs[2 * np_]
        y = refs[2 * np_ + 1]
        acc = jax.lax.dot_general(xs[0][...], ws[0][...],
                                  (((1,), (0,)), ((), ())),
                                  preferred_element_type=jnp.float32)
        for k in range(1, np_):
            acc = acc + jax.lax.dot_general(xs[k][...], ws[k][...],
                                            (((1,), (0,)), ((), ())),
                                            preferred_element_type=jnp.float32)
        acc = acc + b[...]
        if act == "leaky":
            acc = jnp.where(acc >= 0, acc, 0.1 * acc)
        elif act == "sigmoid":
            acc = jax.nn.sigmoid(acc)
        y[...] = acc
        if with_stats:
            st = refs[2 * np_ + 2]

            @pl.when(i == 0)
            def _():
                st[...] = jnp.zeros_like(st)

            st[...] += jnp.concatenate(
                [jnp.sum(acc, 0, keepdims=True),
                 jnp.sum(acc * acc, 0, keepdims=True)], 0)

    in_specs = (
        [pl.BlockSpec((_BR, p.shape[1]), lambda i: (i, 0)) for p in parts]
        + [pl.BlockSpec(w.shape, lambda i: (0, 0)) for w in wts]
        + [pl.BlockSpec((1, dout), lambda i: (0, 0))]
    )
    out_specs = [pl.BlockSpec((_BR, dout), lambda i: (i, 0))]
    out_shape = [jax.ShapeDtypeStruct((rows, dout), jnp.float32)]
    if with_stats:
        out_specs.append(pl.BlockSpec((2, dout), lambda i: (0, 0)))
        out_shape.append(jax.ShapeDtypeStruct((2, dout), jnp.float32))

    res = pl.pallas_call(
        kern, grid=(grid,), in_specs=in_specs,
        out_specs=out_specs, out_shape=out_shape,
    )(*parts, *wts, bias)
    if with_stats:
        return res[0], res[1]
    return res[0]


def _colstats_pair(xa, xb):
    rows, d = xa.shape
    _BR = _pick_br(rows)
    grid = rows // _BR

    def kern(ra, rb, sa, sb):
        i = pl.program_id(0)

        @pl.when(i == 0)
        def _():
            sa[...] = jnp.zeros_like(sa)
            sb[...] = jnp.zeros_like(sb)

        va = ra[...]
        vb = rb[...]
        sa[...] += jnp.concatenate(
            [jnp.sum(va, 0, keepdims=True),
             jnp.sum(va * va, 0, keepdims=True)], 0)
        sb[...] += jnp.concatenate(
            [jnp.sum(vb, 0, keepdims=True),
             jnp.sum(vb * vb, 0, keepdims=True)], 0)

    return pl.pallas_call(
        kern, grid=(grid,),
        in_specs=[pl.BlockSpec((_BR, d), lambda i: (i, 0)),
                  pl.BlockSpec((_BR, d), lambda i: (i, 0))],
        out_specs=[pl.BlockSpec((2, d), lambda i: (0, 0)),
                   pl.BlockSpec((2, d), lambda i: (0, 0))],
        out_shape=[jax.ShapeDtypeStruct((2, d), jnp.float32),
                   jax.ShapeDtypeStruct((2, d), jnp.float32)],
    )(xa, xb)


def _colstats(*parts):
    rows, d = parts[0].shape
    np_ = len(parts)
    _BR = _pick_br(rows)
    grid = rows // _BR

    def kern(*refs):
        i = pl.program_id(0)
        x = refs[0][...]
        for k in range(1, np_):
            x = x + refs[k][...]
        st = refs[np_]

        @pl.when(i == 0)
        def _():
            st[...] = jnp.zeros_like(st)

        st[...] += jnp.concatenate(
            [jnp.sum(x, 0, keepdims=True),
             jnp.sum(x * x, 0, keepdims=True)], 0)

    return pl.pallas_call(
        kern, grid=(grid,),
        in_specs=[pl.BlockSpec((_BR, d), lambda i: (i, 0)) for _ in parts],
        out_specs=pl.BlockSpec((2, d), lambda i: (0, 0)),
        out_shape=jax.ShapeDtypeStruct((2, d), jnp.float32),
    )(*parts)


def _fold(bn, lin, stats_list, part_dims, rows):
    st = jnp.concatenate(stats_list, axis=1)
    mean = st[0] / rows
    var = jnp.maximum(st[1] / rows - mean * mean, 0.0)
    a = bn["g"] * lax.rsqrt(var + 1e-5)
    shift = bn["b"] - mean * a
    wt = lin["W"].T * a[:, None]
    beff = (lin["b"] + shift @ lin["W"].T).reshape(1, -1)
    w_parts = []
    off = 0
    for pd in part_dims:
        w_parts.append(wt[off:off + pd])
        off += pd
    return w_parts, beff


def _mlp3(parts, stats_list, mlp, rows, last_act="none"):
    part_dims = [p.shape[1] for p in parts]
    w0, b0 = _fold(mlp["bn0"], mlp["lin0"], stats_list, part_dims, rows)
    a1, st1 = _fused_mm(parts, w0, b0, "leaky")
    w1, b1 = _fold(mlp["bn1"], mlp["lin1"], [st1], [a1.shape[1]], rows)
    a2, st2 = _fused_mm([a1], w1, b1, "leaky")
    w2, b2 = _fold(mlp["bn2"], mlp["lin2"], [st2], [a2.shape[1]], rows)
    return _fused_mm([a2], w2, b2, last_act if last_act != "none" else "none")


def kernel(x, edge_index, edge_attr, u, batch, params):
    n_nodes, d_node = x.shape
    n_edges, d_edge = edge_attr.shape
    row = edge_index[0]
    col = edge_index[1]

    gather = _make_gather(n_nodes, n_edges, d_node)
    x_st = _colstats(x)
    ea_st = _colstats(edge_attr)

    for name in ("ml1", "ml2", "ml3"):
        p = params[name]
        hid = p["node1"]["lin2"]["b"].shape[0]

        xs, xd = gather(x, row, col)
        xs_st, xd_st = _colstats_pair(xs, xd)

        edge_attr, ea_st = _mlp3([xs, xd, edge_attr],
                                 [xs_st, xd_st, ea_st],
                                 p["edge"], n_edges)

        h, _ = _mlp3([xs, edge_attr], [xs_st, ea_st],
                     p["node1"], n_edges)

        scatter = _make_scatter(n_nodes, n_edges, hid)
        agg0, agg1 = scatter(h, col)
        agg_st = _colstats(agg0, agg1)

        part_dims = [d_node, hid]
        w0, b0 = _fold(p["node2"]["bn0"], p["node2"]["lin0"],
                       [x_st, agg_st], part_dims, n_nodes)
        a1, st1 = _fused_mm([x, agg0, agg1], [w0[0], w0[1], w0[1]],
                            b0, "leaky")
        w1, b1 = _fold(p["node2"]["bn1"], p["node2"]["lin1"],
                       [st1], [a1.shape[1]], n_nodes)
        a2, st2 = _fused_mm([a1], w1, b1, "leaky")
        w2, b2 = _fold(p["node2"]["bn2"], p["node2"]["lin2"],
                       [st2], [a2.shape[1]], n_nodes)
        x, x_st = _fused_mm([a2], w2, b2, "none")

    y_pred = _fused_mm([x], [params["x_linear"]["W"].T],
                       params["x_linear"]["b"].reshape(1, -1),
                       "sigmoid", with_stats=False)
    edge_label_pred = _fused_mm([edge_attr], [params["edge_linear"]["W"].T],
                                params["edge_linear"]["b"].reshape(1, -1),
                                "sigmoid", with_stats=False)
    return (y_pred, edge_label_pred)


# speedup vs baseline: 1.1539x; 1.0806x over previous
import functools

import jax
import jax.numpy as jnp
from jax import lax
from jax.experimental import pallas as pl
from jax.experimental.pallas import tpu as pltpu
from jax.experimental.pallas import tpu_sc as plsc

NC = 2
NS = 16
NW = NC * NS
CH = 128


@functools.lru_cache(maxsize=None)
def _make_gather(n_nodes, n_edges, d):
    n_chunks = n_edges // CH
    nu = 4
    per_w = (n_chunks + NW - 1) // NW
    n_win = (per_w + nu - 1) // nu
    mesh = plsc.VectorSubcoreMesh(core_axis_name="c", subcore_axis_name="s")

    scratch = []
    for _ in range(nu):
        scratch += [pltpu.VMEM((CH,), jnp.int32), pltpu.VMEM((CH,), jnp.int32),
                    pltpu.VMEM((CH, d), jnp.float32),
                    pltpu.VMEM((CH, d), jnp.float32),
                    pltpu.SemaphoreType.DMA, pltpu.SemaphoreType.DMA,
                    pltpu.SemaphoreType.DMA]

    @functools.partial(
        pl.kernel,
        mesh=mesh,
        out_type=[jax.ShapeDtypeStruct((n_edges, d), jnp.float32),
                  jax.ShapeDtypeStruct((n_edges, d), jnp.float32)],
        scratch_types=scratch,
        compiler_params=pltpu.CompilerParams(use_tc_tiling_on_sc=False),
    )
    def gather_k(x_hbm, row_hbm, col_hbm, o_row, o_col, *bufs):
        c = lax.axis_index("c")
        s = lax.axis_index("s")
        wid = s * NC + c
        sets = [bufs[7 * k:7 * k + 7] for k in range(nu)]

        def window(j, carry):
            def chunk(k):
                return (j * nu + k) * NW + wid

            for k in range(nu):
                idx_r, idx_c, buf_r, buf_c, sem_i, sem_g, sem_w = sets[k]
                ch = chunk(k)

                @pl.when(ch < n_chunks)
                def _(ch=ch, idx_r=idx_r, idx_c=idx_c, sem_i=sem_i):
                    base = ch * CH
                    pltpu.async_copy(row_hbm.at[pl.ds(base, CH)], idx_r, sem_i)
                    pltpu.async_copy(col_hbm.at[pl.ds(base, CH)], idx_c, sem_i)

            for k in range(nu):
                idx_r, idx_c, buf_r, buf_c, sem_i, sem_g, sem_w = sets[k]
                ch = chunk(k)

                @pl.when(ch < n_chunks)
                def _(ch=ch, idx_r=idx_r, idx_c=idx_c, buf_r=buf_r,
                      buf_c=buf_c, sem_i=sem_i, sem_g=sem_g):
                    base = ch * CH
                    pltpu.make_async_copy(
                        row_hbm.at[pl.ds(base, CH)], idx_r, sem_i).wait()
                    pltpu.make_async_copy(
                        col_hbm.at[pl.ds(base, CH)], idx_c, sem_i).wait()
                    pltpu.async_copy(x_hbm.at[idx_r], buf_r, sem_g)
                    pltpu.async_copy(x_hbm.at[idx_c], buf_c, sem_g)

            for k in range(nu):
                idx_r, idx_c, buf_r, buf_c, sem_i, sem_g, sem_w = sets[k]
                ch = chunk(k)

                @pl.when(ch < n_chunks)
                def _(ch=ch, idx_r=idx_r, idx_c=idx_c, buf_r=buf_r,
                      buf_c=buf_c, sem_g=sem_g, sem_w=sem_w):
                    base = ch * CH
                    pltpu.make_async_copy(x_hbm.at[idx_r], buf_r, sem_g).wait()
                    pltpu.make_async_copy(x_hbm.at[idx_c], buf_c, sem_g).wait()
                    pltpu.async_copy(buf_r, o_row.at[pl.ds(base, CH)], sem_w)
                    pltpu.async_copy(buf_c, o_col.at[pl.ds(base, CH)], sem_w)

            for k in range(nu):
                idx_r, idx_c, buf_r, buf_c, sem_i, sem_g, sem_w = sets[k]
                ch = chunk(k)

                @pl.when(ch < n_chunks)
                def _(ch=ch, buf_r=buf_r, buf_c=buf_c, sem_w=sem_w):
                    base = ch * CH
                    pltpu.make_async_copy(
                        buf_r, o_row.at[pl.ds(base, CH)], sem_w).wait()
                    pltpu.make_async_copy(
                        buf_c, o_col.at[pl.ds(base, CH)], sem_w).wait()

            return carry

        lax.fori_loop(0, n_win, window, 0)

    return gather_k


@functools.lru_cache(maxsize=None)
def _make_scatter(n_nodes, n_edges, d):
    n_chunks = n_edges // CH
    nu = 2
    per_w = (n_chunks + NW - 1) // NW
    n_win = (per_w + nu - 1) // nu
    lp = min(d, 128)
    n_pass = d // lp
    nr = n_nodes // NS
    zr = 25
    mesh = plsc.VectorSubcoreMesh(core_axis_name="c", subcore_axis_name="s")

    @functools.partial(
        pl.kernel,
        mesh=mesh,
        out_type=[jax.ShapeDtypeStruct((n_nodes, d), jnp.float32),
                  jax.ShapeDtypeStruct((n_nodes, d), jnp.float32)],
        scratch_types=(
            [pltpu.VMEM((zr, lp), jnp.float32),
             pltpu.VMEM_SHARED((n_nodes, lp), jnp.float32)]
            + [pltpu.VMEM((CH,), jnp.int32), pltpu.VMEM((CH, lp), jnp.float32),
               pltpu.SemaphoreType.DMA, pltpu.SemaphoreType.DMA,
               pltpu.SemaphoreType.DMA] * 2
        ),
        compiler_params=pltpu.CompilerParams(use_tc_tiling_on_sc=True),
    )
    def scatter_k(h_hbm, col_hbm, out0, out1, zbuf, acc, *bufs):
        c = lax.axis_index("c")
        s = lax.axis_index("s")
        wid = s * NC + c
        sets = [bufs[5 * k:5 * k + 5] for k in range(nu)]

        def zrow(r, carry):
            for k in range(lp // 16):
                zbuf[r, pl.ds(k * 16, 16)] = jnp.zeros((16,), jnp.float32)
            return carry

        lax.fori_loop(0, zr, zrow, 0)

        for p in range(n_pass):
            for z in range(nr // zr):
                pltpu.sync_copy(zbuf, acc.at[pl.ds(s * nr + z * zr, zr)])
            plsc.subcore_barrier()

            def window(j, carry):
                def chunk(k):
                    return (j * nu + k) * NW + wid

                for k in range(nu):
                    idx_v, hbuf, sem_i, sem_h, sem_s = sets[k]
                    ch = chunk(k)

                    @pl.when(ch < n_chunks)
                    def _(ch=ch, idx_v=idx_v, hbuf=hbuf,
                          sem_i=sem_i, sem_h=sem_h):
                        base = ch * CH
                        pltpu.async_copy(
                            col_hbm.at[pl.ds(base, CH)], idx_v, sem_i)
                        pltpu.async_copy(
                            h_hbm.at[pl.ds(base, CH), pl.ds(p * lp, lp)],
                            hbuf, sem_h)

                for k in range(nu):
                    idx_v, hbuf, sem_i, sem_h, sem_s = sets[k]
                    ch = chunk(k)

                    @pl.when(ch < n_chunks)
                    def _(ch=ch, idx_v=idx_v, hbuf=hbuf, sem_i=sem_i,
                          sem_h=sem_h, sem_s=sem_s):
                        base = ch * CH
                        pltpu.make_async_copy(
                            col_hbm.at[pl.ds(base, CH)], idx_v, sem_i).wait()
                        pltpu.make_async_copy(
                            h_hbm.at[pl.ds(base, CH), pl.ds(p * lp, lp)],
                            hbuf, sem_h).wait()
                        pltpu.async_copy(hbuf, acc.at[idx_v], sem_s, add=True)

                for k in range(nu):
                    idx_v, hbuf, sem_i, sem_h, sem_s = sets[k]
                    ch = chunk(k)

                    @pl.when(ch < n_chunks)
                    def _(ch=ch, idx_v=idx_v, hbuf=hbuf, sem_s=sem_s):
                        pltpu.make_async_copy(
                            hbuf, acc.at[idx_v], sem_s).wait()

                return carry

            lax.fori_loop(0, n_win, window, 0)
            plsc.subcore_barrier()

            for cnt, pred in ((624, s < NS - 1), (640, s == NS - 1)):
                @pl.when(c == 0)
                def _(cnt=cnt, pred=pred):
                    @pl.when(pred)
                    def _():
                        pltpu.sync_copy(
                            acc.at[pl.ds(s * 624, cnt)],
                            out0.at[pl.ds(s * 624, cnt), pl.ds(p * lp, lp)])

                @pl.when(c == 1)
                def _(cnt=cnt, pred=pred):
                    @pl.when(pred)
                    def _():
                        pltpu.sync_copy(
                            acc.at[pl.ds(s * 624, cnt)],
                            out1.at[pl.ds(s * 624, cnt), pl.ds(p * lp, lp)])

            if p + 1 < n_pass:
                plsc.subcore_barrier()

    return scatter_k


def _pick_br(rows):
    if rows <= 16384 and rows % 8 == 0:
        return rows
    for br in (4000, 2000, 1000):
        if rows % br == 0:
            return br
    raise ValueError(rows)


def _fused_mm(parts, wts, bias, act, with_stats=True, head=None):
    rows = parts[0].shape[0]
    dout = wts[0].shape[1]
    np_ = len(parts)
    _BR = _pick_br(rows)
    grid = rows // _BR

    def kern(*refs):
        i = pl.program_id(0)
        xs = refs[:np_]
        ws = refs[np_:2 * np_]
        b = refs[2 * np_]
        pos = 2 * np_ + 1
        if head is not None:
            hw = refs[pos]
            hb = refs[pos + 1]
            pos += 2
        y = refs[pos]
        acc = jax.lax.dot_general(xs[0][...], ws[0][...],
                                  (((1,), (0,)), ((), ())),
                                  preferred_element_type=jnp.float32)
        for k in range(1, np_):
            acc = acc + jax.lax.dot_general(xs[k][...], ws[k][...],
                                            (((1,), (0,)), ((), ())),
                                            preferred_element_type=jnp.float32)
        acc = acc + b[...]
        if act == "leaky":
            acc = jnp.where(acc >= 0, acc, 0.1 * acc)
        elif act == "sigmoid":
            acc = jax.nn.sigmoid(acc)
        y[...] = acc
        if head is not None:
            yh = refs[pos + 1]
            hacc = jax.lax.dot_general(acc, hw[...], (((1,), (0,)), ((), ())),
                                       preferred_element_type=jnp.float32)
            yh[...] = jax.nn.sigmoid(hacc + hb[...])
        if with_stats:
            st = refs[pos + 1 + (1 if head is not None else 0)]

            @pl.when(i == 0)
            def _():
                st[...] = jnp.zeros_like(st)

            st[...] += jnp.concatenate(
                [jnp.sum(acc, 0, keepdims=True),
                 jnp.sum(acc * acc, 0, keepdims=True)], 0)

    in_specs = (
        [pl.BlockSpec((_BR, p.shape[1]), lambda i: (i, 0)) for p in parts]
        + [pl.BlockSpec(w.shape, lambda i: (0, 0)) for w in wts]
        + [pl.BlockSpec((1, dout), lambda i: (0, 0))]
    )
    args = list(parts) + list(wts) + [bias]
    if head is not None:
        hwm, hbm_ = head
        in_specs += [pl.BlockSpec(hwm.shape, lambda i: (0, 0)),
                     pl.BlockSpec((1, hwm.shape[1]), lambda i: (0, 0))]
        args += [hwm, hbm_]
    out_specs = [pl.BlockSpec((_BR, dout), lambda i: (i, 0))]
    out_shape = [jax.ShapeDtypeStruct((rows, dout), jnp.float32)]
    if head is not None:
        out_specs.append(pl.BlockSpec((_BR, head[0].shape[1]),
                                      lambda i: (i, 0)))
        out_shape.append(
            jax.ShapeDtypeStruct((rows, head[0].shape[1]), jnp.float32))
    if with_stats:
        out_specs.append(pl.BlockSpec((2, dout), lambda i: (0, 0)))
        out_shape.append(jax.ShapeDtypeStruct((2, dout), jnp.float32))

    res = pl.pallas_call(
        kern, grid=(grid,), in_specs=in_specs,
        out_specs=out_specs, out_shape=out_shape,
    )(*args)
    if isinstance(res, (list, tuple)):
        if len(res) == 1:
            return res[0]
        return tuple(res)
    return res


def _colstats_pair(xa, xb):
    rows, d = xa.shape
    _BR = _pick_br(rows)
    grid = rows // _BR

    def kern(ra, rb, sa, sb):
        i = pl.program_id(0)

        @pl.when(i == 0)
        def _():
            sa[...] = jnp.zeros_like(sa)
            sb[...] = jnp.zeros_like(sb)

        va = ra[...]
        vb = rb[...]
        sa[...] += jnp.concatenate(
            [jnp.sum(va, 0, keepdims=True),
             jnp.sum(va * va, 0, keepdims=True)], 0)
        sb[...] += jnp.concatenate(
            [jnp.sum(vb, 0, keepdims=True),
             jnp.sum(vb * vb, 0, keepdims=True)], 0)

    return pl.pallas_call(
        kern, grid=(grid,),
        in_specs=[pl.BlockSpec((_BR, d), lambda i: (i, 0)),
                  pl.BlockSpec((_BR, d), lambda i: (i, 0))],
        out_specs=[pl.BlockSpec((2, d), lambda i: (0, 0)),
                   pl.BlockSpec((2, d), lambda i: (0, 0))],
        out_shape=[jax.ShapeDtypeStruct((2, d), jnp.float32),
                   jax.ShapeDtypeStruct((2, d), jnp.float32)],
    )(xa, xb)


def _colstats(*parts):
    rows, d = parts[0].shape
    np_ = len(parts)
    _BR = _pick_br(rows)
    grid = rows // _BR

    def kern(*refs):
        i = pl.program_id(0)
        x = refs[0][...]
        for k in range(1, np_):
            x = x + refs[k][...]
        st = refs[np_]

        @pl.when(i == 0)
        def _():
            st[...] = jnp.zeros_like(st)

        st[...] += jnp.concatenate(
            [jnp.sum(x, 0, keepdims=True),
             jnp.sum(x * x, 0, keepdims=True)], 0)

    return pl.pallas_call(
        kern, grid=(grid,),
        in_specs=[pl.BlockSpec((_BR, d), lambda i: (i, 0)) for _ in parts],
        out_specs=pl.BlockSpec((2, d), lambda i: (0, 0)),
        out_shape=jax.ShapeDtypeStruct((2, d), jnp.float32),
    )(*parts)


def _fold(bn, lin, stats_list, part_dims, rows):
    st = jnp.concatenate(stats_list, axis=1)
    mean = st[0] / rows
    var = jnp.maximum(st[1] / rows - mean * mean, 0.0)
    a = bn["g"] * lax.rsqrt(var + 1e-5)
    shift = bn["b"] - mean * a
    wt = lin["W"].T * a[:, None]
    beff = (lin["b"] + shift @ lin["W"].T).reshape(1, -1)
    w_parts = []
    off = 0
    for pd in part_dims:
        w_parts.append(wt[off:off + pd])
        off += pd
    return w_parts, beff


def _mlp3(parts, stats_list, mlp, rows, head=None):
    part_dims = [p.shape[1] for p in parts]
    w0, b0 = _fold(mlp["bn0"], mlp["lin0"], stats_list, part_dims, rows)
    a1, st1 = _fused_mm(parts, w0, b0, "leaky")
    w1, b1 = _fold(mlp["bn1"], mlp["lin1"], [st1], [a1.shape[1]], rows)
    a2, st2 = _fused_mm([a1], w1, b1, "leaky")
    w2, b2 = _fold(mlp["bn2"], mlp["lin2"], [st2], [a2.shape[1]], rows)
    return _fused_mm([a2], w2, b2, "none", head=head)


def kernel(x, edge_index, edge_attr, u, batch, params):
    n_nodes, d_node = x.shape
    n_edges, d_edge = edge_attr.shape
    row = edge_index[0]
    col = edge_index[1]

    gather = _make_gather(n_nodes, n_edges, d_node)
    x_st = _colstats(x)
    ea_st = _colstats(edge_attr)

    for name in ("ml1", "ml2", "ml3"):
        p = params[name]
        hid = p["node1"]["lin2"]["b"].shape[0]

        xs, xd = gather(x, row, col)
        xs_st, xd_st = _colstats_pair(xs, xd)

        if name == "ml3":
            edge_attr, edge_label_pred, ea_st = _mlp3(
                [xs, xd, edge_attr], [xs_st, xd_st, ea_st], p["edge"],
                n_edges, head=(params["edge_linear"]["W"].T,
                               params["edge_linear"]["b"].reshape(1, -1)))
        else:
            edge_attr, ea_st = _mlp3([xs, xd, edge_attr],
                                     [xs_st, xd_st, ea_st],
                                     p["edge"], n_edges)

        h, _ = _mlp3([xs, edge_attr], [xs_st, ea_st],
                     p["node1"], n_edges)

        scatter = _make_scatter(n_nodes, n_edges, hid)
        agg0, agg1 = scatter(h, col)
        agg_st = _colstats(agg0, agg1)

        part_dims = [d_node, hid]
        w0, b0 = _fold(p["node2"]["bn0"], p["node2"]["lin0"],
                       [x_st, agg_st], part_dims, n_nodes)
        a1, st1 = _fused_mm([x, agg0, agg1], [w0[0], w0[1], w0[1]],
                            b0, "leaky")
        w1, b1 = _fold(p["node2"]["bn1"], p["node2"]["lin1"],
                       [st1], [a1.shape[1]], n_nodes)
        a2, st2 = _fused_mm([a1], w1, b1, "leaky")
        w2, b2 = _fold(p["node2"]["bn2"], p["node2"]["lin2"],
                       [st2], [a2.shape[1]], n_nodes)
        if name == "ml3":
            _, y_pred = _fused_mm(
                [a2], w2, b2, "none", with_stats=False,
                head=(params["x_linear"]["W"].T,
                      params["x_linear"]["b"].reshape(1, -1)))
        else:
            x, x_st = _fused_mm([a2], w2, b2, "none")

    return (y_pred, edge_label_pred)
